# baseline (speedup 1.0000x reference)
import functools

import jax
import jax.numpy as jnp
from jax import lax
from jax.experimental import pallas as pl
from jax.experimental.pallas import tpu as pltpu
from jax.experimental.pallas import tpu_sc as plsc

EMB = 64
HID = 768
EPS = 1e-6

NC = 2
NS = 16
NW = NC * NS

CHUNK = 128


def _sc_gather(idx_2d, table):
    nrow = idx_2d.shape[0]
    rows_per_w = nrow // NW

    mesh = plsc.VectorSubcoreMesh(core_axis_name="c", subcore_axis_name="s")

    @functools.partial(
        pl.kernel,
        out_type=jax.ShapeDtypeStruct((nrow, CHUNK, EMB), jnp.float32),
        mesh=mesh,
        scratch_types=[
            pltpu.VMEM((rows_per_w, CHUNK), jnp.int32),
            pltpu.VMEM((rows_per_w, CHUNK, EMB), jnp.float32),
            pltpu.SemaphoreType.DMA,
        ],
        compiler_params=pltpu.CompilerParams(use_tc_tiling_on_sc=False),
    )
    def k(idx_hbm, table_hbm, out_hbm, idx_v, rows_v, sem):
        wid = lax.axis_index("s") * NC + lax.axis_index("c")
        base = wid * rows_per_w
        pltpu.sync_copy(idx_hbm.at[pl.ds(base, rows_per_w)], idx_v)
        copies = [
            pltpu.async_copy(table_hbm.at[idx_v.at[j]], rows_v.at[j], sem)
            for j in range(rows_per_w)
        ]
        for c in copies:
            c.wait()
        pltpu.sync_copy(rows_v, out_hbm.at[pl.ds(base, rows_per_w)])

    return k(idx_2d, table)


def _tc_post(gathered, pos, gamma, beta, W, b):
    R = gathered.shape[0]
    BLK = 512
    grid = R // BLK
    pos_blocks = pos.shape[0] // BLK

    def body(x_ref, pos_ref, g_ref, bt_ref, w_ref, bias_ref, o_ref):
        x = x_ref[...] + pos_ref[...]
        mean = jnp.mean(x, axis=-1, keepdims=True)
        xc = x - mean
        var = jnp.mean(xc * xc, axis=-1, keepdims=True)
        xn = xc * lax.rsqrt(var + EPS) * g_ref[...] + bt_ref[...]
        o_ref[...] = (
            jnp.dot(xn, w_ref[...], preferred_element_type=jnp.float32)
            + bias_ref[...]
        )

    return pl.pallas_call(
        body,
        grid=(grid,),
        in_specs=[
            pl.BlockSpec((BLK, EMB), lambda i: (i, 0)),
            pl.BlockSpec((BLK, EMB), lambda i: (i % pos_blocks, 0)),
            pl.BlockSpec((1, EMB), lambda i: (0, 0)),
            pl.BlockSpec((1, EMB), lambda i: (0, 0)),
            pl.BlockSpec((EMB, HID), lambda i: (0, 0)),
            pl.BlockSpec((1, HID), lambda i: (0, 0)),
        ],
        out_specs=pl.BlockSpec((BLK, HID), lambda i: (i, 0)),
        out_shape=jax.ShapeDtypeStruct((R, HID), jnp.float32),
    )(gathered, pos, gamma.reshape(1, EMB), beta.reshape(1, EMB), W,
      b.reshape(1, HID))


def kernel(inputs, emb_table, pos_table, gamma, beta, W, b):
    B, L = inputs.shape
    idx = inputs.reshape(B * L // CHUNK, CHUNK)
    gathered = _sc_gather(idx, emb_table).reshape(B * L, EMB)
    pos = lax.dynamic_slice(pos_table, (0, 0), (L, EMB))
    out = _tc_post(gathered, pos, gamma, beta, W, b)
    return out.reshape(B, L, HID)


# speedup vs baseline: 2.5056x; 2.5056x over previous
import functools

import jax
import jax.numpy as jnp
from jax import lax
from jax.experimental import pallas as pl
from jax.experimental.pallas import tpu as pltpu
from jax.experimental.pallas import tpu_sc as plsc

EMB = 64
HID = 768
EPS = 1e-6

NC = 2
NS = 16
NW = NC * NS

TILE = 8


def _sc_gather(idx, table3d):
    rows_per_w = idx.shape[1]
    nrows = NW * rows_per_w

    mesh = plsc.VectorSubcoreMesh(core_axis_name="c", subcore_axis_name="s")

    @functools.partial(
        pl.kernel,
        out_type=jax.ShapeDtypeStruct((nrows, EMB), jnp.float32),
        mesh=mesh,
        scratch_types=[
            pltpu.VMEM((rows_per_w,), jnp.int32),
            pltpu.VMEM((rows_per_w, EMB), jnp.float32),
            pltpu.SemaphoreType.DMA,
        ],
    )
    def k(idx_hbm, table_hbm, out_hbm, idx_v, out_v, sem):
        wid = lax.axis_index("s") * NC + lax.axis_index("c")
        pltpu.sync_copy(idx_hbm.at[wid], idx_v)

        def issue(g, carry):
            vec = idx_v[pl.ds(g * 16, 16)]
            for lane in range(16):
                ji = vec[lane]
                t = lax.shift_right_logical(ji, 3)
                s = lax.bitwise_and(ji, 7)
                pltpu.async_copy(table_hbm.at[t, s], out_v.at[g * 16 + lane],
                                 sem)
            return carry

        lax.fori_loop(0, rows_per_w // 16, issue, 0)

        out_slice = out_hbm.at[pl.ds(wid * rows_per_w, rows_per_w)]
        pltpu.make_async_copy(out_slice, out_v, sem).wait()
        pltpu.sync_copy(out_v, out_slice)

    return k(idx, table3d)


def _tc_post(gathered, pos, gamma, beta, W, b):
    R = gathered.shape[0]
    BLK = 512
    grid = R // BLK
    pos_blocks = pos.shape[0] // BLK

    def body(x_ref, pos_ref, g_ref, bt_ref, w_ref, bias_ref, o_ref):
        x = x_ref[...] + pos_ref[...]
        mean = jnp.mean(x, axis=-1, keepdims=True)
        xc = x - mean
        var = jnp.mean(xc * xc, axis=-1, keepdims=True)
        xn = xc * lax.rsqrt(var + EPS) * g_ref[...] + bt_ref[...]
        o_ref[...] = (
            jnp.dot(xn, w_ref[...], preferred_element_type=jnp.float32)
            + bias_ref[...]
        )

    return pl.pallas_call(
        body,
        grid=(grid,),
        in_specs=[
            pl.BlockSpec((BLK, EMB), lambda i: (i, 0)),
            pl.BlockSpec((BLK, EMB), lambda i: (i % pos_blocks, 0)),
            pl.BlockSpec((1, EMB), lambda i: (0, 0)),
            pl.BlockSpec((1, EMB), lambda i: (0, 0)),
            pl.BlockSpec((EMB, HID), lambda i: (0, 0)),
            pl.BlockSpec((1, HID), lambda i: (0, 0)),
        ],
        out_specs=pl.BlockSpec((BLK, HID), lambda i: (i, 0)),
        out_shape=jax.ShapeDtypeStruct((R, HID), jnp.float32),
    )(gathered, pos, gamma.reshape(1, EMB), beta.reshape(1, EMB), W,
      b.reshape(1, HID))


def kernel(inputs, emb_table, pos_table, gamma, beta, W, b):
    B, L = inputs.shape
    R = B * L
    V = emb_table.shape[0]
    idx = inputs.reshape(NW, R // NW)
    table3d = emb_table.reshape(V // TILE, TILE, EMB)
    gathered = _sc_gather(idx, table3d)
    pos = lax.dynamic_slice(pos_table, (0, 0), (L, EMB))
    out = _tc_post(gathered, pos, gamma, beta, W, b)
    return out.reshape(B, L, HID)


# speedup vs baseline: 2.5468x; 1.0164x over previous
import functools

import jax
import jax.numpy as jnp
from jax import lax
from jax.experimental import pallas as pl
from jax.experimental.pallas import tpu as pltpu
from jax.experimental.pallas import tpu_sc as plsc

EMB = 64
HID = 768
EPS = 1e-6

NC = 2
NS = 16
NW = NC * NS

TILE = 8


def _sc_gather(idx, table3d):
    rows_per_w = idx.shape[1]
    nrows = NW * rows_per_w

    mesh = plsc.VectorSubcoreMesh(core_axis_name="c", subcore_axis_name="s")

    @functools.partial(
        pl.kernel,
        out_type=jax.ShapeDtypeStruct((nrows, EMB), jnp.float32),
        mesh=mesh,
        scratch_types=[
            pltpu.VMEM((rows_per_w,), jnp.int32),
            pltpu.VMEM((rows_per_w, EMB), jnp.float32),
            pltpu.SemaphoreType.DMA,
        ],
    )
    def k(idx_hbm, table_hbm, out_hbm, idx_v, out_v, sem):
        wid = lax.axis_index("s") * NC + lax.axis_index("c")
        pltpu.sync_copy(idx_hbm.at[wid], idx_v)

        def issue(g, carry):
            vec = idx_v[pl.ds(g * 16, 16)]
            for lane in range(16):
                ji = vec[lane]
                t = lax.shift_right_logical(ji, 3)
                s = lax.bitwise_and(ji, 7)
                pltpu.async_copy(table_hbm.at[t, s], out_v.at[g * 16 + lane],
                                 sem)
            return carry

        lax.fori_loop(0, rows_per_w // 16, issue, 0)

        out_slice = out_hbm.at[pl.ds(wid * rows_per_w, rows_per_w)]
        pltpu.make_async_copy(out_slice, out_v, sem).wait()
        pltpu.sync_copy(out_v, out_slice)

    return k(idx, table3d)


def _tc_post(gathered, pos, gamma, beta, W, b):
    R = gathered.shape[0]
    BLK = 1024
    grid = R // BLK
    pos_blocks = pos.shape[0] // BLK

    def body(x_ref, pos_ref, g_ref, bt_ref, w_ref, bias_ref, o_ref):
        x = x_ref[...] + pos_ref[...]
        mean = jnp.mean(x, axis=-1, keepdims=True)
        xc = x - mean
        var = jnp.mean(xc * xc, axis=-1, keepdims=True)
        xn = xc * lax.rsqrt(var + EPS) * g_ref[...] + bt_ref[...]
        o_ref[...] = (
            jnp.dot(xn, w_ref[...], preferred_element_type=jnp.float32)
            + bias_ref[...]
        )

    return pl.pallas_call(
        body,
        grid=(grid,),
        in_specs=[
            pl.BlockSpec((BLK, EMB), lambda i: (i, 0)),
            pl.BlockSpec((BLK, EMB), lambda i: (i % pos_blocks, 0)),
            pl.BlockSpec((1, EMB), lambda i: (0, 0)),
            pl.BlockSpec((1, EMB), lambda i: (0, 0)),
            pl.BlockSpec((EMB, HID), lambda i: (0, 0)),
            pl.BlockSpec((1, HID), lambda i: (0, 0)),
        ],
        out_specs=pl.BlockSpec((BLK, HID), lambda i: (i, 0)),
        out_shape=jax.ShapeDtypeStruct((R, HID), jnp.float32),
    )(gathered, pos, gamma.reshape(1, EMB), beta.reshape(1, EMB), W,
      b.reshape(1, HID))


def kernel(inputs, emb_table, pos_table, gamma, beta, W, b):
    B, L = inputs.shape
    R = B * L
    V = emb_table.shape[0]
    idx = inputs.reshape(NW, R // NW)
    table3d = emb_table.reshape(V // TILE, TILE, EMB)
    gathered = _sc_gather(idx, table3d)
    pos = lax.dynamic_slice(pos_table, (0, 0), (L, EMB))
    out = _tc_post(gathered, pos, gamma, beta, W, b)
    return out.reshape(B, L, HID)


# speedup vs baseline: 2.5744x; 1.0108x over previous
import functools

import jax
import jax.numpy as jnp
from jax import lax
from jax.experimental import pallas as pl
from jax.experimental.pallas import tpu as pltpu
from jax.experimental.pallas import tpu_sc as plsc

EMB = 64
HID = 768
EPS = 1e-6

NC = 2
NS = 16
NW = NC * NS

TILE = 8


def _sc_gather(idx, table3d):
    rows_per_w = idx.shape[1]
    nrows = NW * rows_per_w

    mesh = plsc.VectorSubcoreMesh(core_axis_name="c", subcore_axis_name="s")

    @functools.partial(
        pl.kernel,
        out_type=jax.ShapeDtypeStruct((nrows, EMB), jnp.float32),
        mesh=mesh,
        scratch_types=[
            pltpu.VMEM((rows_per_w,), jnp.int32),
            pltpu.VMEM((rows_per_w, EMB), jnp.float32),
            pltpu.SemaphoreType.DMA,
        ],
    )
    def k(idx_hbm, table_hbm, out_hbm, idx_v, out_v, sem):
        wid = lax.axis_index("s") * NC + lax.axis_index("c")
        pltpu.sync_copy(idx_hbm.at[wid], idx_v)

        def issue(g, carry):
            vec = idx_v[pl.ds(g * 16, 16)]
            for lane in range(16):
                ji = vec[lane]
                t = lax.shift_right_logical(ji, 3)
                s = lax.bitwise_and(ji, 7)
                pltpu.async_copy(table_hbm.at[t, s], out_v.at[g * 16 + lane],
                                 sem)
            return carry

        lax.fori_loop(0, rows_per_w // 16, issue, 0)

        out_slice = out_hbm.at[pl.ds(wid * rows_per_w, rows_per_w)]
        pltpu.make_async_copy(out_slice, out_v, sem).wait()
        pltpu.sync_copy(out_v, out_slice)

    return k(idx, table3d)


def _tc_post(gathered, pos, gamma, beta, W, b):
    R = gathered.shape[0]
    BLK = 2048
    grid = R // BLK
    pos_blocks = pos.shape[0] // BLK

    def body(x_ref, pos_ref, g_ref, bt_ref, w_ref, bias_ref, o_ref):
        x = x_ref[...] + pos_ref[...]
        mean = jnp.mean(x, axis=-1, keepdims=True)
        xc = x - mean
        var = jnp.mean(xc * xc, axis=-1, keepdims=True)
        xn = xc * lax.rsqrt(var + EPS) * g_ref[...] + bt_ref[...]
        o_ref[...] = (
            jnp.dot(xn, w_ref[...], preferred_element_type=jnp.float32)
            + bias_ref[...]
        )

    return pl.pallas_call(
        body,
        grid=(grid,),
        in_specs=[
            pl.BlockSpec((BLK, EMB), lambda i: (i, 0)),
            pl.BlockSpec((BLK, EMB), lambda i: (i % pos_blocks, 0)),
            pl.BlockSpec((1, EMB), lambda i: (0, 0)),
            pl.BlockSpec((1, EMB), lambda i: (0, 0)),
            pl.BlockSpec((EMB, HID), lambda i: (0, 0)),
            pl.BlockSpec((1, HID), lambda i: (0, 0)),
        ],
        out_specs=pl.BlockSpec((BLK, HID), lambda i: (i, 0)),
        out_shape=jax.ShapeDtypeStruct((R, HID), jnp.float32),
    )(gathered, pos, gamma.reshape(1, EMB), beta.reshape(1, EMB), W,
      b.reshape(1, HID))


def kernel(inputs, emb_table, pos_table, gamma, beta, W, b):
    B, L = inputs.shape
    R = B * L
    V = emb_table.shape[0]
    idx = inputs.reshape(NW, R // NW)
    table3d = emb_table.reshape(V // TILE, TILE, EMB)
    gathered = _sc_gather(idx, table3d)
    pos = lax.dynamic_slice(pos_table, (0, 0), (L, EMB))
    out = _tc_post(gathered, pos, gamma, beta, W, b)
    return out.reshape(B, L, HID)
